# baseline (speedup 1.0000x reference)
import functools

import jax
import jax.numpy as jnp
from jax import lax
from jax.experimental import pallas as pl
from jax.experimental.pallas import tpu as pltpu
from jax.experimental.pallas import tpu_sc as plsc

_LANES = 16


@functools.lru_cache(maxsize=None)
def _make_pack(V, D, C=2048):
    nb = -(-V // C)

    def body(a_ref, b_ref, x_ref):
        eye = jnp.eye(D, dtype=jnp.float32)
        dn = (((0,), (0,)), ((), ()))
        x_ref[:, 0:D] = lax.dot_general(
            a_ref[...], eye, dn, precision=lax.Precision.HIGHEST,
            preferred_element_type=jnp.float32)
        x_ref[:, D:2 * D] = lax.dot_general(
            b_ref[...], eye, dn, precision=lax.Precision.HIGHEST,
            preferred_element_type=jnp.float32)

    return pl.pallas_call(
        body,
        grid=(nb,),
        in_specs=[pl.BlockSpec((D, C), lambda i: (0, i)),
                  pl.BlockSpec((D, C), lambda i: (0, i))],
        out_specs=pl.BlockSpec((C, 2 * D), lambda i: (i, 0)),
        out_shape=jax.ShapeDtypeStruct((V, 2 * D), jnp.float32),
    )


@functools.lru_cache(maxsize=None)
def _make_bags(B, L, V, D, NC, NS, weighted):
    NW = NC * NS
    bags_w = B // NW
    GROUP = 16
    rows_per_group = GROUP * L
    n_groups = bags_w // GROUP
    CH = D // _LANES
    n_units = 2 * n_groups

    mesh = plsc.VectorSubcoreMesh(core_axis_name="c", subcore_axis_name="s")

    @functools.partial(
        pl.kernel,
        out_type=jax.ShapeDtypeStruct((B, 2 * D), jnp.float32),
        mesh=mesh,
        scratch_types=[
            pltpu.VMEM((bags_w, L), jnp.int32),
            pltpu.VMEM((bags_w, L), jnp.int32),
            pltpu.VMEM((bags_w, L), jnp.float32),
            pltpu.VMEM((bags_w, L), jnp.float32),
            pltpu.VMEM((rows_per_group, 2 * D), jnp.float32),
            pltpu.VMEM((rows_per_group, 2 * D), jnp.float32),
            pltpu.VMEM((GROUP, D), jnp.float32),
            pltpu.SemaphoreType.DMA,
            pltpu.SemaphoreType.DMA,
        ],
        compiler_params=pltpu.CompilerParams(use_tc_tiling_on_sc=False),
    )
    def k(ia, ib, *rest):
        if weighted:
            (wa, wb, x, out,
             idxa, idxb, wva, wvb, rows0, rows1, outst, sem0, sem1) = rest
        else:
            (x, out,
             idxa, idxb, wva, wvb, rows0, rows1, outst, sem0, sem1) = rest
        wid = lax.axis_index("s") * NC + lax.axis_index("c")
        row0 = wid * bags_w

        pltpu.sync_copy(ia.at[pl.ds(row0, bags_w)], idxa)
        pltpu.sync_copy(ib.at[pl.ds(row0, bags_w)], idxb)
        if weighted:
            pltpu.sync_copy(wa.at[pl.ds(row0, bags_w)], wva)
            pltpu.sync_copy(wb.at[pl.ds(row0, bags_w)], wvb)

        tables = ((idxa, wva, 0), (idxb, wvb, 1))
        units = [tables[t] + (g,) for t in range(2) for g in range(n_groups)]
        rows = (rows0, rows1)
        sems = (sem0, sem1)

        def fire(u):
            idxs, _, t, g = units[u]
            nb = u % 2

            def fb(j, carry, idxs=idxs, g=g, nb=nb):
                pltpu.async_copy(x.at[idxs.at[g * GROUP + j]],
                                 rows[nb].at[pl.ds(j * L, L)], sems[nb])
                return carry

            lax.fori_loop(0, GROUP, fb, 0)

        def drain(u):
            nb = u % 2
            pltpu.make_async_copy(x.at[pl.ds(0, rows_per_group)],
                                  rows[nb], sems[nb]).wait()

        fire(0)
        for u in range(n_units):
            if u + 1 < n_units:
                fire(u + 1)
            drain(u)

            _, wv, t, g = units[u]
            rb = rows[u % 2]
            off = t * D

            def bag_body(j, carry, wv=wv, rb=rb, g=g, off=off):
                r0 = j * L
                bag = g * GROUP + j
                if weighted:
                    w_lo = wv[bag, pl.ds(0, _LANES)]
                    w_hi = wv[bag, pl.ds(L - _LANES, _LANES)]
                accs = [jnp.zeros((_LANES,), jnp.float32) for _ in range(CH)]
                for l in range(L):
                    if weighted:
                        if l < _LANES:
                            src_v, lane = w_lo, l
                        else:
                            src_v, lane = w_hi, l - (L - _LANES)
                        wl = jnp.take_along_axis(
                            src_v, jnp.full((_LANES,), lane, jnp.int32),
                            axis=0)
                    for c in range(CH):
                        r = rb[r0 + l, pl.ds(off + c * _LANES, _LANES)]
                        accs[c] = accs[c] + (r * wl if weighted else r)
                for c in range(CH):
                    outst[j, pl.ds(c * _LANES, _LANES)] = accs[c]
                return carry

            lax.fori_loop(0, GROUP, bag_body, 0)
            pltpu.sync_copy(
                outst,
                out.at[pl.ds(row0 + g * GROUP, GROUP), pl.ds(t * D, D)])

    return k


def kernel(indices_t0, indices_t1, w_indices_t0, w_indices_t1,
           weights_t0, weights_t1, table0, table1, wtable0, wtable1):
    B, L = indices_t0.shape
    V, D = table0.shape
    info = plsc.get_sparse_core_info()
    pack = _make_pack(V, D)
    x01 = pack(jnp.transpose(table0), jnp.transpose(table1))
    xw = pack(jnp.transpose(wtable0), jnp.transpose(wtable1))
    as_i32 = lambda a: a if a.dtype == jnp.int32 else a.astype(jnp.int32)
    bags_u = _make_bags(B, L, V, D, info.num_cores, info.num_subcores, False)
    bags_w = _make_bags(B, L, V, D, info.num_cores, info.num_subcores, True)
    out01 = bags_u(as_i32(indices_t0), as_i32(indices_t1), x01)
    outw = bags_w(as_i32(w_indices_t0), as_i32(w_indices_t1),
                  weights_t0, weights_t1, xw)
    return jnp.concatenate([out01, outw], axis=1)


# speedup vs baseline: 1.2562x; 1.2562x over previous
import functools

import jax
import jax.numpy as jnp
from jax import lax
from jax.experimental import pallas as pl
from jax.experimental.pallas import tpu as pltpu
from jax.experimental.pallas import tpu_sc as plsc

_LANES = 16


@functools.lru_cache(maxsize=None)
def _make_pack(V, D, C=2048):
    nb = -(-V // C)

    def body(a_ref, b_ref, x_ref):
        eye = jnp.eye(D, dtype=jnp.float32)
        dn = (((0,), (0,)), ((), ()))

        def tr(a):
            hi = a.astype(jnp.bfloat16).astype(jnp.float32)
            lo = a - hi
            t_hi = lax.dot_general(hi, eye, dn,
                                   preferred_element_type=jnp.float32)
            t_lo = lax.dot_general(lo, eye, dn,
                                   preferred_element_type=jnp.float32)
            return t_hi + t_lo

        x_ref[:, 0:D] = tr(a_ref[...])
        x_ref[:, D:2 * D] = tr(b_ref[...])

    return pl.pallas_call(
        body,
        grid=(nb,),
        in_specs=[pl.BlockSpec((D, C), lambda i: (0, i)),
                  pl.BlockSpec((D, C), lambda i: (0, i))],
        out_specs=pl.BlockSpec((C, 2 * D), lambda i: (i, 0)),
        out_shape=jax.ShapeDtypeStruct((V, 2 * D), jnp.float32),
    )


@functools.lru_cache(maxsize=None)
def _make_bags(B, L, V, D, NC, NS, weighted):
    NW = NC * NS
    bags_w = B // NW
    GROUP = 16
    rows_per_group = GROUP * L
    n_groups = bags_w // GROUP
    CH = D // _LANES
    n_units = 2 * n_groups

    mesh = plsc.VectorSubcoreMesh(core_axis_name="c", subcore_axis_name="s")

    @functools.partial(
        pl.kernel,
        out_type=jax.ShapeDtypeStruct((B, 2 * D), jnp.float32),
        mesh=mesh,
        scratch_types=[
            pltpu.VMEM((bags_w, L), jnp.int32),
            pltpu.VMEM((bags_w, L), jnp.int32),
            pltpu.VMEM((bags_w, L), jnp.float32),
            pltpu.VMEM((bags_w, L), jnp.float32),
            pltpu.VMEM((rows_per_group, 2 * D), jnp.float32),
            pltpu.VMEM((rows_per_group, 2 * D), jnp.float32),
            pltpu.VMEM((GROUP, D), jnp.float32),
            pltpu.SemaphoreType.DMA,
            pltpu.SemaphoreType.DMA,
        ],
        compiler_params=pltpu.CompilerParams(use_tc_tiling_on_sc=False),
    )
    def k(ia, ib, *rest):
        if weighted:
            (wa, wb, x, out,
             idxa, idxb, wva, wvb, rows0, rows1, outst, sem0, sem1) = rest
        else:
            (x, out,
             idxa, idxb, wva, wvb, rows0, rows1, outst, sem0, sem1) = rest
        wid = lax.axis_index("s") * NC + lax.axis_index("c")
        row0 = wid * bags_w

        pltpu.sync_copy(ia.at[pl.ds(row0, bags_w)], idxa)
        pltpu.sync_copy(ib.at[pl.ds(row0, bags_w)], idxb)
        if weighted:
            pltpu.sync_copy(wa.at[pl.ds(row0, bags_w)], wva)
            pltpu.sync_copy(wb.at[pl.ds(row0, bags_w)], wvb)

        tables = ((idxa, wva, 0), (idxb, wvb, 1))
        units = [tables[t] + (g,) for t in range(2) for g in range(n_groups)]
        rows = (rows0, rows1)
        sems = (sem0, sem1)

        def fire(u):
            idxs, _, t, g = units[u]
            nb = u % 2

            def fb(j, carry, idxs=idxs, g=g, nb=nb):
                pltpu.async_copy(x.at[idxs.at[g * GROUP + j]],
                                 rows[nb].at[pl.ds(j * L, L)], sems[nb])
                return carry

            lax.fori_loop(0, GROUP, fb, 0)

        def drain(u):
            nb = u % 2
            pltpu.make_async_copy(x.at[pl.ds(0, rows_per_group)],
                                  rows[nb], sems[nb]).wait()

        fire(0)
        for u in range(n_units):
            if u + 1 < n_units:
                fire(u + 1)
            drain(u)

            _, wv, t, g = units[u]
            rb = rows[u % 2]
            off = t * D

            def bag_body(j, carry, wv=wv, rb=rb, g=g, off=off):
                r0 = j * L
                bag = g * GROUP + j
                if weighted:
                    w_lo = wv[bag, pl.ds(0, _LANES)]
                    w_hi = wv[bag, pl.ds(L - _LANES, _LANES)]
                accs = [jnp.zeros((_LANES,), jnp.float32) for _ in range(CH)]
                for l in range(L):
                    if weighted:
                        if l < _LANES:
                            src_v, lane = w_lo, l
                        else:
                            src_v, lane = w_hi, l - (L - _LANES)
                        wl = jnp.take_along_axis(
                            src_v, jnp.full((_LANES,), lane, jnp.int32),
                            axis=0)
                    for c in range(CH):
                        r = rb[r0 + l, pl.ds(off + c * _LANES, _LANES)]
                        accs[c] = accs[c] + (r * wl if weighted else r)
                for c in range(CH):
                    outst[j, pl.ds(c * _LANES, _LANES)] = accs[c]
                return carry

            lax.fori_loop(0, GROUP, bag_body, 0)
            pltpu.sync_copy(
                outst,
                out.at[pl.ds(row0 + g * GROUP, GROUP), pl.ds(t * D, D)])

    return k


def kernel(indices_t0, indices_t1, w_indices_t0, w_indices_t1,
           weights_t0, weights_t1, table0, table1, wtable0, wtable1):
    B, L = indices_t0.shape
    V, D = table0.shape
    info = plsc.get_sparse_core_info()
    pack = _make_pack(V, D)
    x01 = pack(jnp.transpose(table0), jnp.transpose(table1))
    xw = pack(jnp.transpose(wtable0), jnp.transpose(wtable1))
    as_i32 = lambda a: a if a.dtype == jnp.int32 else a.astype(jnp.int32)
    bags_u = _make_bags(B, L, V, D, info.num_cores, info.num_subcores, False)
    bags_w = _make_bags(B, L, V, D, info.num_cores, info.num_subcores, True)
    out01 = bags_u(as_i32(indices_t0), as_i32(indices_t1), x01)
    outw = bags_w(as_i32(w_indices_t0), as_i32(w_indices_t1),
                  weights_t0, weights_t1, xw)
    return jnp.concatenate([out01, outw], axis=1)


# speedup vs baseline: 1.3820x; 1.1001x over previous
import functools

import jax
import jax.numpy as jnp
from jax import lax
from jax.experimental import pallas as pl
from jax.experimental.pallas import tpu as pltpu
from jax.experimental.pallas import tpu_sc as plsc

_LANES = 16


@functools.lru_cache(maxsize=None)
def _make_pack(V, D, C=2048):
    nb = -(-V // C)

    def body(a_ref, b_ref, x_ref):
        eye = jnp.eye(D, dtype=jnp.float32)
        eye2 = jnp.concatenate([eye, eye], axis=0)
        dn = (((0,), (0,)), ((), ()))

        def tr(a):
            hi = a.astype(jnp.bfloat16).astype(jnp.float32)
            lo = a - hi
            a2 = jnp.concatenate([hi, lo], axis=0)
            return lax.dot_general(a2, eye2, dn,
                                   preferred_element_type=jnp.float32)

        x_ref[...] = jnp.concatenate([tr(a_ref[...]), tr(b_ref[...])],
                                     axis=1)

    return pl.pallas_call(
        body,
        grid=(nb,),
        in_specs=[pl.BlockSpec((D, C), lambda i: (0, i)),
                  pl.BlockSpec((D, C), lambda i: (0, i))],
        out_specs=pl.BlockSpec((C, 2 * D), lambda i: (i, 0)),
        out_shape=jax.ShapeDtypeStruct((V, 2 * D), jnp.float32),
    )


@functools.lru_cache(maxsize=None)
def _make_bags(B, L, V, D, NC, NS, weighted):
    NW = NC * NS
    bags_w = B // NW
    GROUP = 16
    rows_per_group = GROUP * L
    n_groups = bags_w // GROUP
    CH = D // _LANES
    n_units = 2 * n_groups

    mesh = plsc.VectorSubcoreMesh(core_axis_name="c", subcore_axis_name="s")

    @functools.partial(
        pl.kernel,
        out_type=jax.ShapeDtypeStruct((B, 2 * D), jnp.float32),
        mesh=mesh,
        scratch_types=[
            pltpu.VMEM((bags_w, L), jnp.int32),
            pltpu.VMEM((bags_w, L), jnp.int32),
            pltpu.VMEM((bags_w, L), jnp.float32),
            pltpu.VMEM((bags_w, L), jnp.float32),
            pltpu.VMEM((rows_per_group, 2 * D), jnp.float32),
            pltpu.VMEM((rows_per_group, 2 * D), jnp.float32),
            pltpu.VMEM((GROUP, D), jnp.float32),
            pltpu.SemaphoreType.DMA,
            pltpu.SemaphoreType.DMA,
        ],
        compiler_params=pltpu.CompilerParams(use_tc_tiling_on_sc=False),
    )
    def k(ia, ib, *rest):
        if weighted:
            (wa, wb, x, out,
             idxa, idxb, wva, wvb, rows0, rows1, outst, sem0, sem1) = rest
        else:
            (x, out,
             idxa, idxb, wva, wvb, rows0, rows1, outst, sem0, sem1) = rest
        wid = lax.axis_index("s") * NC + lax.axis_index("c")
        row0 = wid * bags_w

        pltpu.sync_copy(ia.at[pl.ds(row0, bags_w)], idxa)
        pltpu.sync_copy(ib.at[pl.ds(row0, bags_w)], idxb)
        if weighted:
            pltpu.sync_copy(wa.at[pl.ds(row0, bags_w)], wva)
            pltpu.sync_copy(wb.at[pl.ds(row0, bags_w)], wvb)

        tables = ((idxa, wva, 0), (idxb, wvb, 1))
        units = [tables[t] + (g,) for t in range(2) for g in range(n_groups)]
        rows = (rows0, rows1)
        sems = (sem0, sem1)

        def fire(u):
            idxs, _, t, g = units[u]
            nb = u % 2

            def fb(j, carry, idxs=idxs, g=g, nb=nb):
                pltpu.async_copy(x.at[idxs.at[g * GROUP + j]],
                                 rows[nb].at[pl.ds(j * L, L)], sems[nb])
                return carry

            lax.fori_loop(0, GROUP, fb, 0)

        def drain(u):
            nb = u % 2
            pltpu.make_async_copy(x.at[pl.ds(0, rows_per_group)],
                                  rows[nb], sems[nb]).wait()

        fire(0)
        for u in range(n_units):
            if u + 1 < n_units:
                fire(u + 1)
            drain(u)

            _, wv, t, g = units[u]
            rb = rows[u % 2]
            off = t * D

            def bag_body(j, carry, wv=wv, rb=rb, g=g, off=off):
                r0 = j * L
                bag = g * GROUP + j
                if weighted:
                    w_lo = wv[bag, pl.ds(0, _LANES)]
                    w_hi = wv[bag, pl.ds(L - _LANES, _LANES)]
                accs = [jnp.zeros((_LANES,), jnp.float32) for _ in range(CH)]
                for l in range(L):
                    if weighted:
                        if l < _LANES:
                            src_v, lane = w_lo, l
                        else:
                            src_v, lane = w_hi, l - (L - _LANES)
                        wl = jnp.take_along_axis(
                            src_v, jnp.full((_LANES,), lane, jnp.int32),
                            axis=0)
                    for c in range(CH):
                        r = rb[r0 + l, pl.ds(off + c * _LANES, _LANES)]
                        accs[c] = accs[c] + (r * wl if weighted else r)
                for c in range(CH):
                    outst[j, pl.ds(c * _LANES, _LANES)] = accs[c]
                return carry

            lax.fori_loop(0, GROUP, bag_body, 0)
            pltpu.sync_copy(
                outst,
                out.at[pl.ds(row0 + g * GROUP, GROUP), pl.ds(t * D, D)])

    return k


def kernel(indices_t0, indices_t1, w_indices_t0, w_indices_t1,
           weights_t0, weights_t1, table0, table1, wtable0, wtable1):
    B, L = indices_t0.shape
    V, D = table0.shape
    info = plsc.get_sparse_core_info()
    pack = _make_pack(V, D)
    x01 = pack(jnp.transpose(table0), jnp.transpose(table1))
    xw = pack(jnp.transpose(wtable0), jnp.transpose(wtable1))
    as_i32 = lambda a: a if a.dtype == jnp.int32 else a.astype(jnp.int32)
    bags_u = _make_bags(B, L, V, D, info.num_cores, info.num_subcores, False)
    bags_w = _make_bags(B, L, V, D, info.num_cores, info.num_subcores, True)
    out01 = bags_u(as_i32(indices_t0), as_i32(indices_t1), x01)
    outw = bags_w(as_i32(w_indices_t0), as_i32(w_indices_t1),
                  weights_t0, weights_t1, xw)
    return jnp.concatenate([out01, outw], axis=1)


# speedup vs baseline: 1.5681x; 1.1347x over previous
import functools

import jax
import jax.numpy as jnp
from jax import lax
from jax.experimental import pallas as pl
from jax.experimental.pallas import tpu as pltpu
from jax.experimental.pallas import tpu_sc as plsc

_LANES = 16


@functools.lru_cache(maxsize=None)
def _make_pack(V, D, C=2048):
    nb = -(-V // C)

    def body(a_ref, b_ref, x_ref):
        eye = jnp.eye(D, dtype=jnp.float32)
        eye2 = jnp.concatenate([eye, eye], axis=0)
        dn = (((0,), (0,)), ((), ()))

        def tr(a):
            hi = a.astype(jnp.bfloat16).astype(jnp.float32)
            lo = a - hi
            a2 = jnp.concatenate([hi, lo], axis=0)
            return lax.dot_general(a2, eye2, dn,
                                   preferred_element_type=jnp.float32)

        x_ref[...] = jnp.concatenate([tr(a_ref[...]), tr(b_ref[...])],
                                     axis=1)

    return pl.pallas_call(
        body,
        grid=(nb,),
        in_specs=[pl.BlockSpec((D, C), lambda i: (0, i)),
                  pl.BlockSpec((D, C), lambda i: (0, i))],
        out_specs=pl.BlockSpec((C, 2 * D), lambda i: (i, 0)),
        out_shape=jax.ShapeDtypeStruct((V, 2 * D), jnp.float32),
    )


@functools.lru_cache(maxsize=None)
def _make_bags(B, L, V, D, NC, NS, weighted):
    NW = NC * NS
    bags_w = B // NW
    GROUP = 32
    rows_per_group = GROUP * L
    n_groups = bags_w // GROUP
    CH = D // _LANES
    n_units = 2 * n_groups

    mesh = plsc.VectorSubcoreMesh(core_axis_name="c", subcore_axis_name="s")

    @functools.partial(
        pl.kernel,
        out_type=jax.ShapeDtypeStruct((B, 2 * D), jnp.float32),
        mesh=mesh,
        scratch_types=[
            pltpu.VMEM((bags_w, L), jnp.int32),
            pltpu.VMEM((bags_w, L), jnp.int32),
            pltpu.VMEM((bags_w, L), jnp.float32),
            pltpu.VMEM((bags_w, L), jnp.float32),
            pltpu.VMEM((rows_per_group, D), jnp.float32),
            pltpu.VMEM((rows_per_group, D), jnp.float32),
            pltpu.VMEM((GROUP, D), jnp.float32),
            pltpu.SemaphoreType.DMA,
            pltpu.SemaphoreType.DMA,
        ],
        compiler_params=pltpu.CompilerParams(use_tc_tiling_on_sc=False),
    )
    def k(ia, ib, *rest):
        if weighted:
            (wa, wb, x, out,
             idxa, idxb, wva, wvb, rows0, rows1, outst, sem0, sem1) = rest
        else:
            (x, out,
             idxa, idxb, wva, wvb, rows0, rows1, outst, sem0, sem1) = rest
        wid = lax.axis_index("s") * NC + lax.axis_index("c")
        row0 = wid * bags_w

        pltpu.sync_copy(ia.at[pl.ds(row0, bags_w)], idxa)
        pltpu.sync_copy(ib.at[pl.ds(row0, bags_w)], idxb)
        if weighted:
            pltpu.sync_copy(wa.at[pl.ds(row0, bags_w)], wva)
            pltpu.sync_copy(wb.at[pl.ds(row0, bags_w)], wvb)

        def remap(j, carry, idxs=None, t=0):
            lo = idxs[j, pl.ds(0, _LANES)]
            hi = idxs[j, pl.ds(L - _LANES, _LANES)]
            idxs[j, pl.ds(L - _LANES, _LANES)] = hi * 2 + t
            idxs[j, pl.ds(0, _LANES)] = lo * 2 + t
            return carry

        lax.fori_loop(0, bags_w, functools.partial(remap, idxs=idxa, t=0), 0)
        lax.fori_loop(0, bags_w, functools.partial(remap, idxs=idxb, t=1), 0)

        tables = ((idxa, wva, 0), (idxb, wvb, 1))
        units = [tables[t] + (g,) for t in range(2) for g in range(n_groups)]
        rows = (rows0, rows1)
        sems = (sem0, sem1)

        def fire(u):
            idxs, _, t, g = units[u]
            nb = u % 2

            def fb(j, carry, idxs=idxs, g=g, nb=nb):
                pltpu.async_copy(x.at[idxs.at[g * GROUP + j]],
                                 rows[nb].at[pl.ds(j * L, L)], sems[nb])
                return carry

            lax.fori_loop(0, GROUP, fb, 0)

        def drain(u):
            nb = u % 2
            pltpu.make_async_copy(x.at[pl.ds(0, rows_per_group)],
                                  rows[nb], sems[nb]).wait()

        fire(0)
        for u in range(n_units):
            if u + 1 < n_units:
                fire(u + 1)
            drain(u)

            _, wv, t, g = units[u]
            rb = rows[u % 2]
            off = 0

            def bag_body(j, carry, wv=wv, rb=rb, g=g, off=off):
                r0 = j * L
                bag = g * GROUP + j
                if weighted:
                    w_lo = wv[bag, pl.ds(0, _LANES)]
                    w_hi = wv[bag, pl.ds(L - _LANES, _LANES)]
                accs = [jnp.zeros((_LANES,), jnp.float32) for _ in range(CH)]
                for l in range(L):
                    if weighted:
                        if l < _LANES:
                            src_v, lane = w_lo, l
                        else:
                            src_v, lane = w_hi, l - (L - _LANES)
                        wl = jnp.take_along_axis(
                            src_v, jnp.full((_LANES,), lane, jnp.int32),
                            axis=0)
                    for c in range(CH):
                        r = rb[r0 + l, pl.ds(off + c * _LANES, _LANES)]
                        accs[c] = accs[c] + (r * wl if weighted else r)
                for c in range(CH):
                    outst[j, pl.ds(c * _LANES, _LANES)] = accs[c]
                return carry

            lax.fori_loop(0, GROUP, bag_body, 0)
            pltpu.sync_copy(
                outst,
                out.at[pl.ds(row0 + g * GROUP, GROUP), pl.ds(t * D, D)])

    return k


def kernel(indices_t0, indices_t1, w_indices_t0, w_indices_t1,
           weights_t0, weights_t1, table0, table1, wtable0, wtable1):
    B, L = indices_t0.shape
    V, D = table0.shape
    info = plsc.get_sparse_core_info()
    pack = _make_pack(V, D)
    x01 = pack(jnp.transpose(table0), jnp.transpose(table1))
    xw = pack(jnp.transpose(wtable0), jnp.transpose(wtable1))
    as_i32 = lambda a: a if a.dtype == jnp.int32 else a.astype(jnp.int32)
    bags_u = _make_bags(B, L, V, D, info.num_cores, info.num_subcores, False)
    bags_w = _make_bags(B, L, V, D, info.num_cores, info.num_subcores, True)
    x01r = jnp.reshape(x01, (2 * V, D))
    xwr = jnp.reshape(xw, (2 * V, D))
    out01 = bags_u(as_i32(indices_t0), as_i32(indices_t1), x01r)
    outw = bags_w(as_i32(w_indices_t0), as_i32(w_indices_t1),
                  weights_t0, weights_t1, xwr)
    return jnp.concatenate([out01, outw], axis=1)


# speedup vs baseline: 1.8086x; 1.1533x over previous
import functools

import jax
import jax.numpy as jnp
from jax import lax
from jax.experimental import pallas as pl
from jax.experimental.pallas import tpu as pltpu
from jax.experimental.pallas import tpu_sc as plsc

_LANES = 16


@functools.lru_cache(maxsize=None)
def _make_pack(V, D, C=4096):
    nb = -(-V // C)

    def body(a_ref, b_ref, x_ref):
        eye = jnp.eye(D, dtype=jnp.float32)
        eye2 = jnp.concatenate([eye, eye], axis=0)
        dn = (((0,), (0,)), ((), ()))

        def tr(a):
            bits = lax.bitcast_convert_type(a, jnp.int32)
            hi = lax.bitcast_convert_type(
                bits & jnp.int32(-65536), jnp.float32)
            lo = a - hi
            a2 = jnp.concatenate([hi, lo], axis=0)
            return lax.dot_general(a2, eye2, dn,
                                   preferred_element_type=jnp.float32)

        x_ref[...] = jnp.concatenate([tr(a_ref[...]), tr(b_ref[...])],
                                     axis=1)

    return pl.pallas_call(
        body,
        grid=(nb,),
        in_specs=[pl.BlockSpec((D, C), lambda i: (0, i)),
                  pl.BlockSpec((D, C), lambda i: (0, i))],
        out_specs=pl.BlockSpec((C, 2 * D), lambda i: (i, 0)),
        out_shape=jax.ShapeDtypeStruct((V, 2 * D), jnp.float32),
    )


@functools.lru_cache(maxsize=None)
def _make_bags(B, L, V, D, NC, NS, weighted):
    NW = NC * NS
    bags_w = B // NW
    GROUP = 32
    rows_per_group = GROUP * L
    n_groups = bags_w // GROUP
    CH = D // _LANES
    n_units = 2 * n_groups

    mesh = plsc.VectorSubcoreMesh(core_axis_name="c", subcore_axis_name="s")

    @functools.partial(
        pl.kernel,
        out_type=jax.ShapeDtypeStruct((B, 2 * D), jnp.float32),
        mesh=mesh,
        scratch_types=[
            pltpu.VMEM((bags_w, L), jnp.int32),
            pltpu.VMEM((bags_w, L), jnp.int32),
            pltpu.VMEM((bags_w, L), jnp.float32),
            pltpu.VMEM((bags_w, L), jnp.float32),
            pltpu.VMEM((rows_per_group, D), jnp.float32),
            pltpu.VMEM((rows_per_group, D), jnp.float32),
            pltpu.VMEM((GROUP, D), jnp.float32),
            pltpu.SemaphoreType.DMA,
            pltpu.SemaphoreType.DMA,
        ],
        compiler_params=pltpu.CompilerParams(use_tc_tiling_on_sc=False),
    )
    def k(ia, ib, *rest):
        if weighted:
            (wa, wb, x, out,
             idxa, idxb, wva, wvb, rows0, rows1, outst, sem0, sem1) = rest
        else:
            (x, out,
             idxa, idxb, wva, wvb, rows0, rows1, outst, sem0, sem1) = rest
        wid = lax.axis_index("s") * NC + lax.axis_index("c")
        row0 = wid * bags_w

        pltpu.sync_copy(ia.at[pl.ds(row0, bags_w)], idxa)
        pltpu.sync_copy(ib.at[pl.ds(row0, bags_w)], idxb)
        if weighted:
            pltpu.sync_copy(wa.at[pl.ds(row0, bags_w)], wva)
            pltpu.sync_copy(wb.at[pl.ds(row0, bags_w)], wvb)

        def remap(j, carry, idxs=None, t=0):
            lo = idxs[j, pl.ds(0, _LANES)]
            hi = idxs[j, pl.ds(L - _LANES, _LANES)]
            idxs[j, pl.ds(L - _LANES, _LANES)] = hi * 2 + t
            idxs[j, pl.ds(0, _LANES)] = lo * 2 + t
            return carry

        lax.fori_loop(0, bags_w, functools.partial(remap, idxs=idxa, t=0), 0)
        lax.fori_loop(0, bags_w, functools.partial(remap, idxs=idxb, t=1), 0)

        tables = ((idxa, wva, 0), (idxb, wvb, 1))
        units = [tables[t] + (g,) for t in range(2) for g in range(n_groups)]
        rows = (rows0, rows1)
        sems = (sem0, sem1)

        def fire(u):
            idxs, _, t, g = units[u]
            nb = u % 2

            def fb(j, carry, idxs=idxs, g=g, nb=nb):
                pltpu.async_copy(x.at[idxs.at[g * GROUP + j]],
                                 rows[nb].at[pl.ds(j * L, L)], sems[nb])
                return carry

            lax.fori_loop(0, GROUP, fb, 0)

        def drain(u):
            nb = u % 2
            pltpu.make_async_copy(x.at[pl.ds(0, rows_per_group)],
                                  rows[nb], sems[nb]).wait()

        fire(0)
        for u in range(n_units):
            if u + 1 < n_units:
                fire(u + 1)
            drain(u)

            _, wv, t, g = units[u]
            rb = rows[u % 2]
            off = 0

            def bag_body(j, carry, wv=wv, rb=rb, g=g, off=off):
                r0 = j * L
                bag = g * GROUP + j
                if weighted:
                    w_lo = wv[bag, pl.ds(0, _LANES)]
                    w_hi = wv[bag, pl.ds(L - _LANES, _LANES)]
                accs = [jnp.zeros((_LANES,), jnp.float32) for _ in range(CH)]
                for l in range(L):
                    if weighted:
                        if l < _LANES:
                            src_v, lane = w_lo, l
                        else:
                            src_v, lane = w_hi, l - (L - _LANES)
                        wl = jnp.take_along_axis(
                            src_v, jnp.full((_LANES,), lane, jnp.int32),
                            axis=0)
                    for c in range(CH):
                        r = rb[r0 + l, pl.ds(off + c * _LANES, _LANES)]
                        accs[c] = accs[c] + (r * wl if weighted else r)
                for c in range(CH):
                    outst[j, pl.ds(c * _LANES, _LANES)] = accs[c]
                return carry

            lax.fori_loop(0, GROUP, bag_body, 0)
            pltpu.sync_copy(
                outst,
                out.at[pl.ds(row0 + g * GROUP, GROUP), pl.ds(t * D, D)])

    return k


def kernel(indices_t0, indices_t1, w_indices_t0, w_indices_t1,
           weights_t0, weights_t1, table0, table1, wtable0, wtable1):
    B, L = indices_t0.shape
    V, D = table0.shape
    info = plsc.get_sparse_core_info()
    pack = _make_pack(V, D)
    x01 = pack(jnp.transpose(table0), jnp.transpose(table1))
    xw = pack(jnp.transpose(wtable0), jnp.transpose(wtable1))
    as_i32 = lambda a: a if a.dtype == jnp.int32 else a.astype(jnp.int32)
    bags_u = _make_bags(B, L, V, D, info.num_cores, info.num_subcores, False)
    bags_w = _make_bags(B, L, V, D, info.num_cores, info.num_subcores, True)
    x01r = jnp.reshape(x01, (2 * V, D))
    xwr = jnp.reshape(xw, (2 * V, D))
    out01 = bags_u(as_i32(indices_t0), as_i32(indices_t1), x01r)
    outw = bags_w(as_i32(w_indices_t0), as_i32(w_indices_t1),
                  weights_t0, weights_t1, xwr)
    return jnp.concatenate([out01, outw], axis=1)


# speedup vs baseline: 1.9061x; 1.0539x over previous
import functools

import jax
import jax.numpy as jnp
from jax import lax
from jax.experimental import pallas as pl
from jax.experimental.pallas import tpu as pltpu
from jax.experimental.pallas import tpu_sc as plsc

_LANES = 16


@functools.lru_cache(maxsize=None)
def _make_pack(V, D, C=4096):
    nb = -(-V // C)

    def body(a_ref, b_ref, x_ref):
        eye = jnp.eye(D, dtype=jnp.float32)
        eye2 = jnp.concatenate([eye, eye], axis=0)
        dn = (((0,), (0,)), ((), ()))

        def tr(a):
            bits = lax.bitcast_convert_type(a, jnp.int32)
            hi = lax.bitcast_convert_type(
                bits & jnp.int32(-65536), jnp.float32)
            lo = a - hi
            a2 = jnp.concatenate([hi, lo], axis=0)
            return lax.dot_general(a2, eye2, dn,
                                   preferred_element_type=jnp.float32)

        x_ref[...] = jnp.concatenate([tr(a_ref[...]), tr(b_ref[...])],
                                     axis=1)

    return pl.pallas_call(
        body,
        grid=(nb,),
        in_specs=[pl.BlockSpec((D, C), lambda i: (0, i)),
                  pl.BlockSpec((D, C), lambda i: (0, i))],
        out_specs=pl.BlockSpec((C, 2 * D), lambda i: (i, 0)),
        out_shape=jax.ShapeDtypeStruct((V, 2 * D), jnp.float32),
    )


@functools.lru_cache(maxsize=None)
def _make_prep(B, L, weighted, C=2048):
    nb = B // C
    n_in = 4 if weighted else 2

    def body(*refs):
        (ia_ref, ib_ref), rest = refs[:2], refs[2:]
        out_ref = rest[-1]
        parts = [jnp.transpose(ia_ref[...]) * 2,
                 jnp.transpose(ib_ref[...]) * 2 + 1]
        if weighted:
            wa_ref, wb_ref = rest[0], rest[1]
            parts.append(lax.bitcast_convert_type(
                jnp.transpose(wa_ref[...]), jnp.int32))
            parts.append(lax.bitcast_convert_type(
                jnp.transpose(wb_ref[...]), jnp.int32))
        pad = jnp.zeros((C, 128 - L * len(parts)), jnp.int32)
        out_ref[...] = jnp.concatenate(parts + [pad], axis=1)

    in_specs = [pl.BlockSpec((L, C), lambda i: (0, i))
                for _ in range(n_in)]
    return pl.pallas_call(
        body,
        grid=(nb,),
        in_specs=in_specs,
        out_specs=pl.BlockSpec((C, 128), lambda i: (i, 0)),
        out_shape=jax.ShapeDtypeStruct((B, 128), jnp.int32),
    )


@functools.lru_cache(maxsize=None)
def _make_bags(B, L, V, D, NC, NS, weighted):
    NW = NC * NS
    bags_w = B // NW
    GROUP = 16
    rpb = 2 * L
    rows_per_group = GROUP * rpb
    n_groups = bags_w // GROUP
    CH = D // _LANES

    mesh = plsc.VectorSubcoreMesh(core_axis_name="c", subcore_axis_name="s")

    @functools.partial(
        pl.kernel,
        out_type=jax.ShapeDtypeStruct((B, 2 * D), jnp.float32),
        mesh=mesh,
        scratch_types=[
            pltpu.VMEM((bags_w, 2 * L), jnp.int32),
            pltpu.VMEM((bags_w, 2 * L), jnp.int32),
            pltpu.VMEM((rows_per_group, D), jnp.float32),
            pltpu.VMEM((rows_per_group, D), jnp.float32),
            pltpu.VMEM((GROUP, 2 * D), jnp.float32),
            pltpu.SemaphoreType.DMA,
            pltpu.SemaphoreType.DMA,
        ],
        compiler_params=pltpu.CompilerParams(use_tc_tiling_on_sc=False,
                                             needs_layout_passes=False),
    )
    def k(meta, x, out, idxab, wvab, rows0, rows1, outst, sem0, sem1):
        wid = lax.axis_index("s") * NC + lax.axis_index("c")
        row0 = wid * bags_w

        pltpu.sync_copy(meta.at[pl.ds(row0, bags_w), pl.ds(0, 2 * L)],
                        idxab)
        if weighted:
            pltpu.sync_copy(meta.at[pl.ds(row0, bags_w),
                                    pl.ds(2 * L, 2 * L)], wvab)

        rows = (rows0, rows1)
        sems = (sem0, sem1)

        def fire(g):
            nb = g % 2

            def fb(j, carry, g=g, nb=nb):
                pltpu.async_copy(
                    x.at[idxab.at[g * GROUP + j]],
                    rows[nb].at[pl.ds(j * rpb, rpb)], sems[nb])
                return carry

            lax.fori_loop(0, GROUP, fb, 0)

        def drain(g):
            nb = g % 2
            pltpu.make_async_copy(x.at[pl.ds(0, rows_per_group)],
                                  rows[nb], sems[nb]).wait()

        fire(0)
        for g in range(n_groups):
            if g + 1 < n_groups:
                fire(g + 1)
            drain(g)
            rb = rows[g % 2]

            def bag_body(j, carry, rb=rb, g=g):
                bag = g * GROUP + j
                for t in range(2):
                    r0 = j * rpb + t * L
                    if weighted:
                        w_lo = plsc.bitcast(
                            wvab[bag, pl.ds(t * L, _LANES)], jnp.float32)
                        w_hi = plsc.bitcast(
                            wvab[bag, pl.ds(t * L + L - _LANES, _LANES)],
                            jnp.float32)
                    accs = [jnp.zeros((_LANES,), jnp.float32)
                            for _ in range(CH)]
                    for l in range(L):
                        if weighted:
                            if l < _LANES:
                                src_v, lane = w_lo, l
                            else:
                                src_v, lane = w_hi, l - (L - _LANES)
                            wl = jnp.take_along_axis(
                                src_v,
                                jnp.full((_LANES,), lane, jnp.int32),
                                axis=0)
                        for c in range(CH):
                            r = rb[r0 + l, pl.ds(c * _LANES, _LANES)]
                            accs[c] = accs[c] + (r * wl if weighted else r)
                    for c in range(CH):
                        outst[j, pl.ds(t * D + c * _LANES, _LANES)] = accs[c]
                return carry

            lax.fori_loop(0, GROUP, bag_body, 0)
            pltpu.sync_copy(outst,
                            out.at[pl.ds(row0 + g * GROUP, GROUP)])

    return k


def kernel(indices_t0, indices_t1, w_indices_t0, w_indices_t1,
           weights_t0, weights_t1, table0, table1, wtable0, wtable1):
    B, L = indices_t0.shape
    V, D = table0.shape
    info = plsc.get_sparse_core_info()
    as_i32 = lambda a: a if a.dtype == jnp.int32 else a.astype(jnp.int32)
    tr = jnp.transpose

    pack = _make_pack(V, D)
    x01 = pack(tr(table0), tr(table1))
    xw = pack(tr(wtable0), tr(wtable1))
    meta_u = _make_prep(B, L, False)(tr(as_i32(indices_t0)),
                                     tr(as_i32(indices_t1)))
    meta_w = _make_prep(B, L, True)(tr(as_i32(w_indices_t0)),
                                    tr(as_i32(w_indices_t1)),
                                    tr(weights_t0), tr(weights_t1))
    bags_u = _make_bags(B, L, V, D, info.num_cores, info.num_subcores, False)
    bags_w = _make_bags(B, L, V, D, info.num_cores, info.num_subcores, True)
    out01 = bags_u(meta_u, jnp.reshape(x01, (2 * V, D)))
    outw = bags_w(meta_w, jnp.reshape(xw, (2 * V, D)))
    return jnp.concatenate([out01, outw], axis=1)


# speedup vs baseline: 2.0198x; 1.0596x over previous
import functools

import jax
import jax.numpy as jnp
from jax import lax
from jax.experimental import pallas as pl
from jax.experimental.pallas import tpu as pltpu
from jax.experimental.pallas import tpu_sc as plsc

_LANES = 16


@functools.lru_cache(maxsize=None)
def _make_pack(V, D, C=8192):
    nb = -(-V // C)

    def body(a_ref, b_ref, x_ref):
        eye = jnp.eye(D, dtype=jnp.float32)
        eye2 = jnp.concatenate([eye, eye], axis=0)
        dn = (((0,), (0,)), ((), ()))

        def tr(a):
            bits = lax.bitcast_convert_type(a, jnp.int32)
            hi = lax.bitcast_convert_type(
                bits & jnp.int32(-65536), jnp.float32)
            lo = a - hi
            a2 = jnp.concatenate([hi, lo], axis=0)
            return lax.dot_general(a2, eye2, dn,
                                   preferred_element_type=jnp.float32)

        x_ref[...] = jnp.concatenate([tr(a_ref[...]), tr(b_ref[...])],
                                     axis=1)

    return pl.pallas_call(
        body,
        grid=(nb,),
        in_specs=[pl.BlockSpec((D, C), lambda i: (0, i)),
                  pl.BlockSpec((D, C), lambda i: (0, i))],
        out_specs=pl.BlockSpec((C, 2 * D), lambda i: (i, 0)),
        out_shape=jax.ShapeDtypeStruct((V, 2 * D), jnp.float32),
    )


@functools.lru_cache(maxsize=None)
def _make_prep(B, L, weighted, C=2048):
    nb = B // C
    n_in = 4 if weighted else 2

    def body(*refs):
        (ia_ref, ib_ref), rest = refs[:2], refs[2:]
        out_ref = rest[-1]
        parts = [jnp.transpose(ia_ref[...]) * 2,
                 jnp.transpose(ib_ref[...]) * 2 + 1]
        if weighted:
            wa_ref, wb_ref = rest[0], rest[1]
            parts.append(lax.bitcast_convert_type(
                jnp.transpose(wa_ref[...]), jnp.int32))
            parts.append(lax.bitcast_convert_type(
                jnp.transpose(wb_ref[...]), jnp.int32))
        pad = jnp.zeros((C, 128 - L * len(parts)), jnp.int32)
        out_ref[...] = jnp.concatenate(parts + [pad], axis=1)

    in_specs = [pl.BlockSpec((L, C), lambda i: (0, i))
                for _ in range(n_in)]
    return pl.pallas_call(
        body,
        grid=(nb,),
        in_specs=in_specs,
        out_specs=pl.BlockSpec((C, 128), lambda i: (i, 0)),
        out_shape=jax.ShapeDtypeStruct((B, 128), jnp.int32),
    )


@functools.lru_cache(maxsize=None)
def _make_bags(B, L, V, D, NC, NS, weighted):
    NW = NC * NS
    bags_w = B // NW
    GROUP = 16
    rpb = 2 * L
    rows_per_group = GROUP * rpb
    n_groups = bags_w // GROUP
    CH = D // _LANES

    mesh = plsc.VectorSubcoreMesh(core_axis_name="c", subcore_axis_name="s")

    @functools.partial(
        pl.kernel,
        out_type=jax.ShapeDtypeStruct((B, 2 * D), jnp.float32),
        mesh=mesh,
        scratch_types=[
            pltpu.VMEM((bags_w, 2 * L), jnp.int32),
            pltpu.VMEM((bags_w, 2 * L), jnp.int32),
            pltpu.VMEM((rows_per_group, D), jnp.float32),
            pltpu.VMEM((rows_per_group, D), jnp.float32),
            pltpu.VMEM((GROUP, 2 * D), jnp.float32),
            pltpu.SemaphoreType.DMA,
            pltpu.SemaphoreType.DMA,
        ],
        compiler_params=pltpu.CompilerParams(use_tc_tiling_on_sc=False,
                                             needs_layout_passes=False),
    )
    def k(meta, x, out, idxab, wvab, rows0, rows1, outst, sem0, sem1):
        wid = lax.axis_index("s") * NC + lax.axis_index("c")
        row0 = wid * bags_w

        pltpu.sync_copy(meta.at[pl.ds(row0, bags_w), pl.ds(0, 2 * L)],
                        idxab)
        if weighted:
            pltpu.sync_copy(meta.at[pl.ds(row0, bags_w),
                                    pl.ds(2 * L, 2 * L)], wvab)

        rows = (rows0, rows1)
        sems = (sem0, sem1)

        def fire(g):
            nb = g % 2

            def fb(j, carry, g=g, nb=nb):
                pltpu.async_copy(
                    x.at[idxab.at[g * GROUP + j]],
                    rows[nb].at[pl.ds(j * rpb, rpb)], sems[nb])
                return carry

            lax.fori_loop(0, GROUP, fb, 0)

        def drain(g):
            nb = g % 2
            pltpu.make_async_copy(x.at[pl.ds(0, rows_per_group)],
                                  rows[nb], sems[nb]).wait()

        fire(0)
        for g in range(n_groups):
            if g + 1 < n_groups:
                fire(g + 1)
            drain(g)
            rb = rows[g % 2]

            def bag_body(j, carry, rb=rb, g=g):
                bag = g * GROUP + j
                for t in range(2):
                    r0 = j * rpb + t * L
                    if weighted:
                        w_lo = plsc.bitcast(
                            wvab[bag, pl.ds(t * L, _LANES)], jnp.float32)
                        w_hi = plsc.bitcast(
                            wvab[bag, pl.ds(t * L + L - _LANES, _LANES)],
                            jnp.float32)
                    accs = [jnp.zeros((_LANES,), jnp.float32)
                            for _ in range(CH)]
                    for l in range(L):
                        if weighted:
                            if l < _LANES:
                                src_v, lane = w_lo, l
                            else:
                                src_v, lane = w_hi, l - (L - _LANES)
                            wl = jnp.take_along_axis(
                                src_v,
                                jnp.full((_LANES,), lane, jnp.int32),
                                axis=0)
                        for c in range(CH):
                            r = rb[r0 + l, pl.ds(c * _LANES, _LANES)]
                            accs[c] = accs[c] + (r * wl if weighted else r)
                    for c in range(CH):
                        outst[j, pl.ds(t * D + c * _LANES, _LANES)] = accs[c]
                return carry

            lax.fori_loop(0, GROUP, bag_body, 0)
            pltpu.sync_copy(outst,
                            out.at[pl.ds(row0 + g * GROUP, GROUP)])

    return k


def kernel(indices_t0, indices_t1, w_indices_t0, w_indices_t1,
           weights_t0, weights_t1, table0, table1, wtable0, wtable1):
    B, L = indices_t0.shape
    V, D = table0.shape
    info = plsc.get_sparse_core_info()
    as_i32 = lambda a: a if a.dtype == jnp.int32 else a.astype(jnp.int32)
    tr = jnp.transpose

    pack = _make_pack(V, D)
    x01 = pack(tr(table0), tr(table1))
    xw = pack(tr(wtable0), tr(wtable1))
    meta_u = _make_prep(B, L, False)(tr(as_i32(indices_t0)),
                                     tr(as_i32(indices_t1)))
    meta_w = _make_prep(B, L, True)(tr(as_i32(w_indices_t0)),
                                    tr(as_i32(w_indices_t1)),
                                    tr(weights_t0), tr(weights_t1))
    bags_u = _make_bags(B, L, V, D, info.num_cores, info.num_subcores, False)
    bags_w = _make_bags(B, L, V, D, info.num_cores, info.num_subcores, True)
    out01 = bags_u(meta_u, jnp.reshape(x01, (2 * V, D)))
    outw = bags_w(meta_w, jnp.reshape(xw, (2 * V, D)))
    return jnp.concatenate([out01, outw], axis=1)
